# baseline (speedup 1.0000x reference)
import functools

import jax
import jax.numpy as jnp
from jax import lax
from jax.experimental import pallas as pl
from jax.experimental.pallas import tpu as pltpu
from jax.experimental.pallas import tpu_sc as plsc

N = 10000
NPAD = 10240
E = 320000
D = 128
WIDTH = 144
N_GRAPHS = 64
ALPHA = 0.2
NCORES = 2
NSUB = 16
NW = NCORES * NSUB
EPW = E // NW
CHUNK = 40
NCHUNK = EPW // CHUNK
NSEG = 125
CPS = NCHUNK // NSEG
ROWS_PER_SUB = NPAD // NSUB


def _leaky(x):
    return jnp.where(x >= 0, x, ALPHA * x)


def _prep_tables(h, Wm, Asrc, Adst, srcT_ref, edT_ref, cvec_ref):
    Wh = jnp.dot(h, Wm, preferred_element_type=jnp.float32)
    es = jnp.dot(Wh, Asrc, preferred_element_type=jnp.float32)
    ed = jnp.dot(Wh, Adst, preferred_element_type=jnp.float32)
    H = es.shape[1]
    srcT_ref[...] = jnp.zeros((NPAD, WIDTH), jnp.float32)
    srcT_ref[0:N, 0:D] = Wh
    srcT_ref[0:N, D:D + H] = es
    edT_ref[...] = jnp.zeros((NPAD, 16), jnp.float32)
    edT_ref[0:N, 0:H] = ed
    cmax = _leaky(jnp.max(es, axis=0, keepdims=True)
                  + jnp.max(ed, axis=0, keepdims=True))
    cvec_ref[...] = jnp.full((1, 16), jnp.inf, jnp.float32)
    cvec_ref[0:1, 0:H] = cmax


def _emb_prep_body(x_ref, We_ref, be_ref, Wm_ref, As_ref, Ad_ref,
                   h_ref, srcT_ref, edT_ref, cvec_ref):
    h = jnp.dot(x_ref[...], We_ref[...],
                preferred_element_type=jnp.float32) + be_ref[...]
    h_ref[...] = h
    _prep_tables(h, Wm_ref[...], As_ref[...], Ad_ref[...],
                 srcT_ref, edT_ref, cvec_ref)


def _finish_prep_body(part_ref, h_ref, nn_ref, R_ref, Wm_ref, As_ref, Ad_ref,
                      hn_ref, srcT_ref, edT_ref, cvec_ref, H=8):
    acc = part_ref[0] + part_ref[1]
    num = acc[0:N, 0:D]
    den = acc[0:N, D:D + H]
    den_exp = jnp.dot(den, R_ref[...], preferred_element_type=jnp.float32)
    out = num / (den_exp + 1e-16)
    g = out * nn_ref[...]
    act = jnp.where(g > 0, g, jnp.exp(g) - 1.0)
    hn = act + h_ref[...]
    hn_ref[...] = hn
    _prep_tables(hn, Wm_ref[...], As_ref[...], Ad_ref[...],
                 srcT_ref, edT_ref, cvec_ref)


def _finish_pool_body(part_ref, h_ref, nn_ref, R_ref, ind_ref,
                      Wm1_ref, bm1_ref, Wm2_ref, bm2_ref, Wm3_ref, bm3_ref,
                      y_ref):
    H = 1
    acc = part_ref[0] + part_ref[1]
    num = acc[0:N, 0:D]
    den = acc[0:N, D:D + H]
    den_exp = jnp.dot(den, R_ref[...], preferred_element_type=jnp.float32)
    out = num / (den_exp + 1e-16)
    g = out * nn_ref[...]
    act = jnp.where(g > 0, g, jnp.exp(g) - 1.0)
    h = act + h_ref[...]
    iota = lax.broadcasted_iota(jnp.int32, (N, N_GRAPHS), 1)
    onehot = (ind_ref[...] == iota).astype(jnp.float32)
    counts = jnp.sum(onehot, axis=0, keepdims=True)
    sums = lax.dot_general(onehot, h, (((0,), (0,)), ((), ())),
                           preferred_element_type=jnp.float32)
    pooled = sums / jnp.maximum(counts, 1.0).reshape(N_GRAPHS, 1)
    y1 = jnp.maximum(jnp.dot(pooled, Wm1_ref[...],
                             preferred_element_type=jnp.float32)
                     + bm1_ref[...], 0.0)
    y2 = jnp.maximum(jnp.dot(y1, Wm2_ref[...],
                             preferred_element_type=jnp.float32)
                     + bm2_ref[...], 0.0)
    y_ref[...] = jnp.dot(y2, Wm3_ref[...],
                         preferred_element_type=jnp.float32) + bm3_ref[...]


def _sc_edge_body(srcT, edT, cvh, srch, dsth, zerosh, out_hbm,
                  sidxA, didxA, sidxB, didxB, sdidxA, sdidxB,
                  srowsA, srowsB, edrowsA, edrowsB,
                  orowsA, orowsB, cvec, acc,
                  semGA, semGB, semSA, semSB, semIA, semIB, H=8):
    HID = D // H
    c = lax.axis_index("c")
    s = lax.axis_index("s")
    wid = s * NCORES + c
    pltpu.sync_copy(zerosh.at[pl.ds(s * ROWS_PER_SUB, ROWS_PER_SUB)],
                    acc.at[pl.ds(s * ROWS_PER_SUB, ROWS_PER_SUB)])
    pltpu.sync_copy(cvh, cvec)
    plsc.subcore_barrier()

    def copy_idx(ci, sidx, didx):
        base = wid * EPW + ci * CHUNK
        pltpu.sync_copy(srch.at[pl.ds(base, CHUNK)], sidx)
        pltpu.sync_copy(dsth.at[pl.ds(base, CHUNK)], didx)

    def gather(sidx, didx, srows, edrows, sem):
        pltpu.async_copy(srcT.at[sidx], srows, sem)
        pltpu.async_copy(edT.at[didx], edrows, sem)

    def wait_gather(srows, edrows, sem):
        pltpu.make_async_copy(srcT.at[pl.ds(0, CHUNK)], srows, sem).wait()
        pltpu.make_async_copy(edT.at[pl.ds(0, CHUNK)], edrows, sem).wait()

    def wait_scatter(orows, sem):
        pltpu.make_async_copy(srcT.at[pl.ds(0, CHUNK)], orows, sem).wait()

    def scatter(orows, sdidx, sem):
        pltpu.async_copy(orows, acc.at[sdidx], sem, add=True)

    def compute(srows, edrows, orows):
        def edge_body(i, carry2):
            es = srows[i, pl.ds(D, 16)]
            ed = edrows[i, :]
            z = es + ed
            e = jnp.where(z >= 0, z, ALPHA * z)
            ex = jnp.exp(e - cvec[...])
            orows[i, pl.ds(D, 16)] = ex
            for h in range(H):
                sc = ex[h]
                scv = lax.broadcast_in_dim(sc, (16,), ())
                for j in range(HID // 16):
                    off = h * HID + j * 16
                    orows[i, pl.ds(off, 16)] = srows[i, pl.ds(off, 16)] * scv
            return carry2

        lax.fori_loop(0, CHUNK, edge_body, 0, unroll=4)

    copy_idx(0, sidxA, didxA)
    gather(sidxA, didxA, srowsA, edrowsA, semGA)

    def pair_body(p, c2):
        ci = p * 2
        copy_idx(ci + 1, sidxB, didxB)
        gather(sidxB, didxB, srowsB, edrowsB, semGB)
        wait_gather(srowsA, edrowsA, semGA)

        @pl.when(p > 0)
        def _():
            wait_scatter(orowsA, semSA)

        pltpu.async_copy(dsth.at[pl.ds(wid * EPW + ci * CHUNK, CHUNK)],
                         sdidxA, semIA)
        compute(srowsA, edrowsA, orowsA)
        pltpu.make_async_copy(dsth.at[pl.ds(0, CHUNK)], sdidxA, semIA).wait()
        scatter(orowsA, sdidxA, semSA)

        @pl.when(p < NCHUNK // 2 - 1)
        def _():
            copy_idx(ci + 2, sidxA, didxA)
            gather(sidxA, didxA, srowsA, edrowsA, semGA)

        wait_gather(srowsB, edrowsB, semGB)

        @pl.when(p > 0)
        def _():
            wait_scatter(orowsB, semSB)

        pltpu.async_copy(dsth.at[pl.ds(wid * EPW + (ci + 1) * CHUNK, CHUNK)],
                         sdidxB, semIB)
        compute(srowsB, edrowsB, orowsB)
        pltpu.make_async_copy(dsth.at[pl.ds(0, CHUNK)], sdidxB, semIB).wait()
        scatter(orowsB, sdidxB, semSB)
        return c2

    lax.fori_loop(0, NCHUNK // 2, pair_body, 0, unroll=False)
    wait_scatter(orowsA, semSA)
    wait_scatter(orowsB, semSB)
    plsc.subcore_barrier()
    pltpu.sync_copy(acc.at[pl.ds(s * ROWS_PER_SUB, ROWS_PER_SUB)],
                    out_hbm.at[c, pl.ds(s * ROWS_PER_SUB, ROWS_PER_SUB)])


def _make_sc_edge(H):
    mesh = plsc.VectorSubcoreMesh(core_axis_name="c", subcore_axis_name="s")
    return functools.partial(
        pl.kernel,
        functools.partial(_sc_edge_body, H=H),
        mesh=mesh,
        compiler_params=pltpu.CompilerParams(use_tc_tiling_on_sc=False),
        out_type=jax.ShapeDtypeStruct((NCORES, NPAD, WIDTH), jnp.float32),
        scratch_types=[
            pltpu.VMEM((CHUNK,), jnp.int32),
            pltpu.VMEM((CHUNK,), jnp.int32),
            pltpu.VMEM((CHUNK,), jnp.int32),
            pltpu.VMEM((CHUNK,), jnp.int32),
            pltpu.VMEM((CHUNK,), jnp.int32),
            pltpu.VMEM((CHUNK,), jnp.int32),
            pltpu.VMEM((CHUNK, WIDTH), jnp.float32),
            pltpu.VMEM((CHUNK, WIDTH), jnp.float32),
            pltpu.VMEM((CHUNK, 16), jnp.float32),
            pltpu.VMEM((CHUNK, 16), jnp.float32),
            pltpu.VMEM((CHUNK, WIDTH), jnp.float32),
            pltpu.VMEM((CHUNK, WIDTH), jnp.float32),
            pltpu.VMEM((16,), jnp.float32),
            pltpu.VMEM_SHARED((NPAD, WIDTH), jnp.float32),
            pltpu.SemaphoreType.DMA,
            pltpu.SemaphoreType.DMA,
            pltpu.SemaphoreType.DMA,
            pltpu.SemaphoreType.DMA,
            pltpu.SemaphoreType.DMA,
            pltpu.SemaphoreType.DMA,
        ],
    )()


def _block_diag(a):
    H = a.shape[0]
    eye = jnp.eye(H, dtype=a.dtype)
    return (a[:, :, None] * eye[:, None, :]).reshape(H * a.shape[1], H)


def kernel(x, edge_index, indicator, n_norm, graph_index, W_emb, b_emb,
           W0, asrc0, adst0, W1, asrc1, adst1, W2, asrc2, adst2,
           W3, asrc3, adst3, Wm1, bm1, Wm2, bm2, Wm3, bm3):
    f32 = jnp.float32
    src = edge_index[0]
    dst = edge_index[1]
    zeros_pad = jnp.zeros((NPAD, WIDTH), f32)

    layer_W = [W0.reshape(D, D), W1.reshape(D, D), W2.reshape(D, D),
               W3.reshape(D, D)]
    layer_As = [_block_diag(asrc0), _block_diag(asrc1), _block_diag(asrc2),
                _block_diag(asrc3)]
    layer_Ad = [_block_diag(adst0), _block_diag(adst1), _block_diag(adst2),
                _block_diag(adst3)]
    R8 = jnp.kron(jnp.eye(8, dtype=f32), jnp.ones((1, 16), f32))
    R1 = jnp.ones((1, D), f32)

    table_shapes = (
        jax.ShapeDtypeStruct((N, D), f32),
        jax.ShapeDtypeStruct((NPAD, WIDTH), f32),
        jax.ShapeDtypeStruct((NPAD, 16), f32),
        jax.ShapeDtypeStruct((1, 16), f32),
    )

    h, srcT, edT, cvec = pl.pallas_call(
        _emb_prep_body, out_shape=table_shapes,
    )(x, W_emb, b_emb.reshape(1, D), layer_W[0], layer_As[0], layer_Ad[0])

    sc_edge8 = _make_sc_edge(8)
    sc_edge1 = _make_sc_edge(1)

    for i in range(3):
        part = (sc_edge8 if i < 3 else sc_edge1)(
            srcT, edT, cvec.reshape(16), src, dst, zeros_pad)
        h, srcT, edT, cvec = pl.pallas_call(
            functools.partial(_finish_prep_body, H=8),
            out_shape=table_shapes,
        )(part, h, n_norm, R8, layer_W[i + 1], layer_As[i + 1],
          layer_Ad[i + 1])

    part = sc_edge1(srcT, edT, cvec.reshape(16), src, dst, zeros_pad)
    y = pl.pallas_call(
        _finish_pool_body,
        out_shape=jax.ShapeDtypeStruct((N_GRAPHS, 10), f32),
    )(part, h, n_norm, R1, indicator.reshape(N, 1),
      Wm1, bm1.reshape(1, -1), Wm2, bm2.reshape(1, -1),
      Wm3, bm3.reshape(1, -1))
    return y


# speedup vs baseline: 1.6101x; 1.6101x over previous
import functools

import jax
import jax.numpy as jnp
from jax import lax
from jax.experimental import pallas as pl
from jax.experimental.pallas import tpu as pltpu
from jax.experimental.pallas import tpu_sc as plsc

N = 10000
NPAD = 10240
E = 320000
D = 128
WIDTH = 144
N_GRAPHS = 64
ALPHA = 0.2
NCORES = 2
NSUB = 16
NW = NCORES * NSUB
EPW = E // NW
CHUNK = 40
NCHUNK = EPW // CHUNK
NSEG = 125
CPS = NCHUNK // NSEG
ROWS_PER_SUB = NPAD // NSUB


def _leaky(x):
    return jnp.where(x >= 0, x, ALPHA * x)


def _prep_tables(h, Wm, Asrc, Adst, srcT_ref, edT_ref, cvec_ref):
    Wh = jnp.dot(h, Wm, preferred_element_type=jnp.float32)
    es = jnp.dot(Wh, Asrc, preferred_element_type=jnp.float32)
    ed = jnp.dot(Wh, Adst, preferred_element_type=jnp.float32)
    H = es.shape[1]
    srcT_ref[...] = jnp.zeros((NPAD, WIDTH), jnp.float32)
    srcT_ref[0:N, 0:D] = Wh
    srcT_ref[0:N, D:D + H] = es
    edT_ref[...] = jnp.zeros((NPAD, 16), jnp.float32)
    edT_ref[0:N, 0:H] = ed
    cmax = _leaky(jnp.max(es, axis=0, keepdims=True)
                  + jnp.max(ed, axis=0, keepdims=True))
    cvec_ref[...] = jnp.full((1, 16), jnp.inf, jnp.float32)
    cvec_ref[0:1, 0:H] = cmax


def _emb_prep_body(x_ref, We_ref, be_ref, Wm_ref, As_ref, Ad_ref,
                   h_ref, srcT_ref, edT_ref, cvec_ref):
    h = jnp.dot(x_ref[...], We_ref[...],
                preferred_element_type=jnp.float32) + be_ref[...]
    h_ref[...] = h
    _prep_tables(h, Wm_ref[...], As_ref[...], Ad_ref[...],
                 srcT_ref, edT_ref, cvec_ref)


def _finish_prep_body(part_ref, h_ref, nn_ref, R_ref, Wm_ref, As_ref, Ad_ref,
                      hn_ref, srcT_ref, edT_ref, cvec_ref, H=8):
    acc = part_ref[0] + part_ref[1]
    num = acc[0:N, 0:D]
    den = acc[0:N, D:D + H]
    den_exp = jnp.dot(den, R_ref[...], preferred_element_type=jnp.float32)
    out = num / (den_exp + 1e-16)
    g = out * nn_ref[...]
    act = jnp.where(g > 0, g, jnp.exp(g) - 1.0)
    hn = act + h_ref[...]
    hn_ref[...] = hn
    _prep_tables(hn, Wm_ref[...], As_ref[...], Ad_ref[...],
                 srcT_ref, edT_ref, cvec_ref)


def _finish_pool_body(part_ref, h_ref, nn_ref, R_ref, ind_ref,
                      Wm1_ref, bm1_ref, Wm2_ref, bm2_ref, Wm3_ref, bm3_ref,
                      y_ref):
    H = 1
    acc = part_ref[0] + part_ref[1]
    num = acc[0:N, 0:D]
    den = acc[0:N, D:D + H]
    den_exp = jnp.dot(den, R_ref[...], preferred_element_type=jnp.float32)
    out = num / (den_exp + 1e-16)
    g = out * nn_ref[...]
    act = jnp.where(g > 0, g, jnp.exp(g) - 1.0)
    h = act + h_ref[...]
    iota = lax.broadcasted_iota(jnp.int32, (N, N_GRAPHS), 1)
    onehot = (ind_ref[...] == iota).astype(jnp.float32)
    counts = jnp.sum(onehot, axis=0, keepdims=True)
    sums = lax.dot_general(onehot, h, (((0,), (0,)), ((), ())),
                           preferred_element_type=jnp.float32)
    pooled = sums / jnp.maximum(counts, 1.0).reshape(N_GRAPHS, 1)
    y1 = jnp.maximum(jnp.dot(pooled, Wm1_ref[...],
                             preferred_element_type=jnp.float32)
                     + bm1_ref[...], 0.0)
    y2 = jnp.maximum(jnp.dot(y1, Wm2_ref[...],
                             preferred_element_type=jnp.float32)
                     + bm2_ref[...], 0.0)
    y_ref[...] = jnp.dot(y2, Wm3_ref[...],
                         preferred_element_type=jnp.float32) + bm3_ref[...]


def _sc_edge_body(srcT, edT, cvh, srch, dsth, zerosh, out_hbm,
                  sidxA, didxA, sidxB, didxB, sdidxA, sdidxB,
                  srowsA, srowsB, edrowsA, edrowsB,
                  orowsA, orowsB, cvec, acc,
                  semGA, semGB, semSA, semSB, semIA, semIB, H=8):
    HID = D // H
    c = lax.axis_index("c")
    s = lax.axis_index("s")
    wid = s * NCORES + c
    pltpu.sync_copy(zerosh.at[pl.ds(s * ROWS_PER_SUB, ROWS_PER_SUB)],
                    acc.at[pl.ds(s * ROWS_PER_SUB, ROWS_PER_SUB)])
    pltpu.sync_copy(cvh, cvec)
    plsc.subcore_barrier()

    def copy_idx(ci, sidx, didx):
        base = wid * EPW + ci * CHUNK
        pltpu.sync_copy(srch.at[pl.ds(base, CHUNK)], sidx)
        pltpu.sync_copy(dsth.at[pl.ds(base, CHUNK)], didx)

    def gather(sidx, didx, srows, edrows, sem):
        pltpu.async_copy(srcT.at[sidx], srows, sem)
        pltpu.async_copy(edT.at[didx], edrows, sem)

    def wait_gather(srows, edrows, sem):
        pltpu.make_async_copy(srcT.at[pl.ds(0, CHUNK)], srows, sem).wait()
        pltpu.make_async_copy(edT.at[pl.ds(0, CHUNK)], edrows, sem).wait()

    def wait_scatter(orows, sem):
        pltpu.make_async_copy(srcT.at[pl.ds(0, CHUNK)], orows, sem).wait()

    def scatter(orows, sdidx, sem):
        pltpu.async_copy(orows, acc.at[sdidx], sem, add=True)

    def compute(srows, edrows, orows):
        def edge_body(i, carry2):
            es = srows[i, pl.ds(D, 16)]
            ed = edrows[i, :]
            z = es + ed
            e = jnp.where(z >= 0, z, ALPHA * z)
            ex = jnp.exp(e - cvec[...])
            orows[i, pl.ds(D, 16)] = ex
            for h in range(H):
                sc = ex[h]
                scv = lax.broadcast_in_dim(sc, (16,), ())
                for j in range(HID // 16):
                    off = h * HID + j * 16
                    orows[i, pl.ds(off, 16)] = srows[i, pl.ds(off, 16)] * scv
            return carry2

        lax.fori_loop(0, CHUNK, edge_body, 0, unroll=False)

    copy_idx(0, sidxA, didxA)
    gather(sidxA, didxA, srowsA, edrowsA, semGA)

    def pair_body(p, c2):
        ci = p * 2
        copy_idx(ci + 1, sidxB, didxB)
        gather(sidxB, didxB, srowsB, edrowsB, semGB)
        wait_gather(srowsA, edrowsA, semGA)

        @pl.when(p > 0)
        def _():
            wait_scatter(orowsA, semSA)

        pltpu.async_copy(dsth.at[pl.ds(wid * EPW + ci * CHUNK, CHUNK)],
                         sdidxA, semIA)
        compute(srowsA, edrowsA, orowsA)
        pltpu.make_async_copy(dsth.at[pl.ds(0, CHUNK)], sdidxA, semIA).wait()
        scatter(orowsA, sdidxA, semSA)

        @pl.when(p < NCHUNK // 2 - 1)
        def _():
            copy_idx(ci + 2, sidxA, didxA)
            gather(sidxA, didxA, srowsA, edrowsA, semGA)

        wait_gather(srowsB, edrowsB, semGB)

        @pl.when(p > 0)
        def _():
            wait_scatter(orowsB, semSB)

        pltpu.async_copy(dsth.at[pl.ds(wid * EPW + (ci + 1) * CHUNK, CHUNK)],
                         sdidxB, semIB)
        compute(srowsB, edrowsB, orowsB)
        pltpu.make_async_copy(dsth.at[pl.ds(0, CHUNK)], sdidxB, semIB).wait()
        scatter(orowsB, sdidxB, semSB)
        return c2

    lax.fori_loop(0, NCHUNK // 2, pair_body, 0, unroll=False)
    wait_scatter(orowsA, semSA)
    wait_scatter(orowsB, semSB)
    plsc.subcore_barrier()
    pltpu.sync_copy(acc.at[pl.ds(s * ROWS_PER_SUB, ROWS_PER_SUB)],
                    out_hbm.at[c, pl.ds(s * ROWS_PER_SUB, ROWS_PER_SUB)])


def _make_sc_edge(H):
    mesh = plsc.VectorSubcoreMesh(core_axis_name="c", subcore_axis_name="s")
    return functools.partial(
        pl.kernel,
        functools.partial(_sc_edge_body, H=H),
        mesh=mesh,
        compiler_params=pltpu.CompilerParams(use_tc_tiling_on_sc=False),
        out_type=jax.ShapeDtypeStruct((NCORES, NPAD, WIDTH), jnp.float32),
        scratch_types=[
            pltpu.VMEM((CHUNK,), jnp.int32),
            pltpu.VMEM((CHUNK,), jnp.int32),
            pltpu.VMEM((CHUNK,), jnp.int32),
            pltpu.VMEM((CHUNK,), jnp.int32),
            pltpu.VMEM((CHUNK,), jnp.int32),
            pltpu.VMEM((CHUNK,), jnp.int32),
            pltpu.VMEM((CHUNK, WIDTH), jnp.float32),
            pltpu.VMEM((CHUNK, WIDTH), jnp.float32),
            pltpu.VMEM((CHUNK, 16), jnp.float32),
            pltpu.VMEM((CHUNK, 16), jnp.float32),
            pltpu.VMEM((CHUNK, WIDTH), jnp.float32),
            pltpu.VMEM((CHUNK, WIDTH), jnp.float32),
            pltpu.VMEM((16,), jnp.float32),
            pltpu.VMEM_SHARED((NPAD, WIDTH), jnp.float32),
            pltpu.SemaphoreType.DMA,
            pltpu.SemaphoreType.DMA,
            pltpu.SemaphoreType.DMA,
            pltpu.SemaphoreType.DMA,
            pltpu.SemaphoreType.DMA,
            pltpu.SemaphoreType.DMA,
        ],
    )()


def _block_diag(a):
    H = a.shape[0]
    eye = jnp.eye(H, dtype=a.dtype)
    return (a[:, :, None] * eye[:, None, :]).reshape(H * a.shape[1], H)


def kernel(x, edge_index, indicator, n_norm, graph_index, W_emb, b_emb,
           W0, asrc0, adst0, W1, asrc1, adst1, W2, asrc2, adst2,
           W3, asrc3, adst3, Wm1, bm1, Wm2, bm2, Wm3, bm3):
    f32 = jnp.float32
    src = edge_index[0]
    dst = edge_index[1]
    zeros_pad = jnp.zeros((NPAD, WIDTH), f32)

    layer_W = [W0.reshape(D, D), W1.reshape(D, D), W2.reshape(D, D),
               W3.reshape(D, D)]
    layer_As = [_block_diag(asrc0), _block_diag(asrc1), _block_diag(asrc2),
                _block_diag(asrc3)]
    layer_Ad = [_block_diag(adst0), _block_diag(adst1), _block_diag(adst2),
                _block_diag(adst3)]
    R8 = jnp.kron(jnp.eye(8, dtype=f32), jnp.ones((1, 16), f32))
    R1 = jnp.ones((1, D), f32)

    table_shapes = (
        jax.ShapeDtypeStruct((N, D), f32),
        jax.ShapeDtypeStruct((NPAD, WIDTH), f32),
        jax.ShapeDtypeStruct((NPAD, 16), f32),
        jax.ShapeDtypeStruct((1, 16), f32),
    )

    h, srcT, edT, cvec = pl.pallas_call(
        _emb_prep_body, out_shape=table_shapes,
    )(x, W_emb, b_emb.reshape(1, D), layer_W[0], layer_As[0], layer_Ad[0])

    sc_edge8 = _make_sc_edge(8)
    sc_edge1 = _make_sc_edge(1)

    for i in range(3):
        part = (sc_edge8 if i < 3 else sc_edge1)(
            srcT, edT, cvec.reshape(16), src, dst, zeros_pad)
        h, srcT, edT, cvec = pl.pallas_call(
            functools.partial(_finish_prep_body, H=8),
            out_shape=table_shapes,
        )(part, h, n_norm, R8, layer_W[i + 1], layer_As[i + 1],
          layer_Ad[i + 1])

    part = sc_edge1(srcT, edT, cvec.reshape(16), src, dst, zeros_pad)
    y = pl.pallas_call(
        _finish_pool_body,
        out_shape=jax.ShapeDtypeStruct((N_GRAPHS, 10), f32),
    )(part, h, n_norm, R1, indicator.reshape(N, 1),
      Wm1, bm1.reshape(1, -1), Wm2, bm2.reshape(1, -1),
      Wm3, bm3.reshape(1, -1))
    return y


# speedup vs baseline: 1.9395x; 1.2046x over previous
import functools

import jax
import jax.numpy as jnp
from jax import lax
from jax.experimental import pallas as pl
from jax.experimental.pallas import tpu as pltpu
from jax.experimental.pallas import tpu_sc as plsc

N = 10000
NPAD = 10240
E = 320000
D = 128
WIDTH = 144
N_GRAPHS = 64
ALPHA = 0.2
NCORES = 2
NSUB = 16
NW = NCORES * NSUB
EPW = E // NW
CHUNK = 40
NCHUNK = EPW // CHUNK
NSEG = 125
CPS = NCHUNK // NSEG
ROWS_PER_SUB = NPAD // NSUB


def _leaky(x):
    return jnp.where(x >= 0, x, ALPHA * x)


def _prep_tables(h, Wm, Asrc, Adst, srcT_ref, edT_ref, cvec_ref):
    Wh = jnp.dot(h, Wm, preferred_element_type=jnp.float32)
    es = jnp.dot(Wh, Asrc, preferred_element_type=jnp.float32)
    ed = jnp.dot(Wh, Adst, preferred_element_type=jnp.float32)
    H = es.shape[1]
    srcT_ref[...] = jnp.zeros((NPAD, WIDTH), jnp.float32)
    srcT_ref[0:N, 0:D] = Wh
    srcT_ref[0:N, D:D + H] = es
    edT_ref[...] = jnp.zeros((NPAD, 16), jnp.float32)
    edT_ref[0:N, 0:H] = ed
    cmax = _leaky(jnp.max(es, axis=0, keepdims=True)
                  + jnp.max(ed, axis=0, keepdims=True))
    cvec_ref[...] = jnp.full((1, 16), jnp.inf, jnp.float32)
    cvec_ref[0:1, 0:H] = cmax


def _emb_prep_body(x_ref, We_ref, be_ref, Wm_ref, As_ref, Ad_ref,
                   h_ref, srcT_ref, edT_ref, cvec_ref):
    h = jnp.dot(x_ref[...], We_ref[...],
                preferred_element_type=jnp.float32) + be_ref[...]
    h_ref[...] = h
    _prep_tables(h, Wm_ref[...], As_ref[...], Ad_ref[...],
                 srcT_ref, edT_ref, cvec_ref)


def _finish_prep_body(part_ref, h_ref, nn_ref, R_ref, Wm_ref, As_ref, Ad_ref,
                      hn_ref, srcT_ref, edT_ref, cvec_ref, H=8):
    acc = part_ref[0] + part_ref[1]
    num = acc[0:N, 0:D]
    den = acc[0:N, D:D + H]
    den_exp = jnp.dot(den, R_ref[...], preferred_element_type=jnp.float32)
    out = num / (den_exp + 1e-16)
    g = out * nn_ref[...]
    act = jnp.where(g > 0, g, jnp.exp(g) - 1.0)
    hn = act + h_ref[...]
    hn_ref[...] = hn
    _prep_tables(hn, Wm_ref[...], As_ref[...], Ad_ref[...],
                 srcT_ref, edT_ref, cvec_ref)


def _finish_pool_body(part_ref, h_ref, nn_ref, R_ref, ind_ref,
                      Wm1_ref, bm1_ref, Wm2_ref, bm2_ref, Wm3_ref, bm3_ref,
                      y_ref):
    H = 1
    acc = part_ref[0] + part_ref[1]
    num = acc[0:N, 0:D]
    den = acc[0:N, D:D + H]
    den_exp = jnp.dot(den, R_ref[...], preferred_element_type=jnp.float32)
    out = num / (den_exp + 1e-16)
    g = out * nn_ref[...]
    act = jnp.where(g > 0, g, jnp.exp(g) - 1.0)
    h = act + h_ref[...]
    iota = lax.broadcasted_iota(jnp.int32, (N, N_GRAPHS), 1)
    onehot = (ind_ref[...] == iota).astype(jnp.float32)
    counts = jnp.sum(onehot, axis=0, keepdims=True)
    sums = lax.dot_general(onehot, h, (((0,), (0,)), ((), ())),
                           preferred_element_type=jnp.float32)
    pooled = sums / jnp.maximum(counts, 1.0).reshape(N_GRAPHS, 1)
    y1 = jnp.maximum(jnp.dot(pooled, Wm1_ref[...],
                             preferred_element_type=jnp.float32)
                     + bm1_ref[...], 0.0)
    y2 = jnp.maximum(jnp.dot(y1, Wm2_ref[...],
                             preferred_element_type=jnp.float32)
                     + bm2_ref[...], 0.0)
    y_ref[...] = jnp.dot(y2, Wm3_ref[...],
                         preferred_element_type=jnp.float32) + bm3_ref[...]


def _sc_edge_body(srcT, edT, cvh, srch, dsth, zerosh, out_hbm,
                  sdA, sdB, sdidxA, sdidxB,
                  srowsA, srowsB, edrowsA, edrowsB,
                  orowsA, orowsB, cvec, acc,
                  semGA, semGB, semSA, semSB, semIA, semIB, H=8):
    HID = D // H
    c = lax.axis_index("c")
    s = lax.axis_index("s")
    wid = s * NCORES + c
    pltpu.sync_copy(zerosh.at[pl.ds(s * ROWS_PER_SUB, ROWS_PER_SUB)],
                    acc.at[pl.ds(s * ROWS_PER_SUB, ROWS_PER_SUB)])
    pltpu.sync_copy(cvh, cvec)
    plsc.subcore_barrier()

    def copy_idx(ci, sdbuf):
        base = (wid * NCHUNK + ci) * 2 * CHUNK
        pltpu.sync_copy(srch.at[pl.ds(base, 2 * CHUNK)], sdbuf)

    def gather(sdbuf, srows, edrows, sem):
        pltpu.async_copy(srcT.at[sdbuf.at[pl.ds(0, CHUNK)]], srows, sem)
        pltpu.async_copy(edT.at[sdbuf.at[pl.ds(CHUNK, CHUNK)]], edrows, sem)

    def wait_gather(srows, edrows, sem):
        pltpu.make_async_copy(srcT.at[pl.ds(0, CHUNK)], srows, sem).wait()
        pltpu.make_async_copy(edT.at[pl.ds(0, CHUNK)], edrows, sem).wait()

    def wait_scatter(orows, sem):
        pltpu.make_async_copy(srcT.at[pl.ds(0, CHUNK)], orows, sem).wait()

    def scatter(orows, sdidx, sem):
        pltpu.async_copy(orows, acc.at[sdidx], sem, add=True)

    def compute(srows, edrows, orows):
        def edge_body(i, carry2):
            es = srows[i, pl.ds(D, 16)]
            ed = edrows[i, :]
            z = es + ed
            e = jnp.where(z >= 0, z, ALPHA * z)
            ex = jnp.exp(e - cvec[...])
            orows[i, pl.ds(D, 16)] = ex
            for h in range(H):
                sc = ex[h]
                scv = lax.broadcast_in_dim(sc, (16,), ())
                for j in range(HID // 16):
                    off = h * HID + j * 16
                    orows[i, pl.ds(off, 16)] = srows[i, pl.ds(off, 16)] * scv
            return carry2

        lax.fori_loop(0, CHUNK, edge_body, 0, unroll=False)

    copy_idx(0, sdA)
    gather(sdA, srowsA, edrowsA, semGA)

    def pair_body(p, c2):
        ci = p * 2
        copy_idx(ci + 1, sdB)
        gather(sdB, srowsB, edrowsB, semGB)
        wait_gather(srowsA, edrowsA, semGA)

        @pl.when(p > 0)
        def _():
            wait_scatter(orowsA, semSA)

        pltpu.async_copy(dsth.at[pl.ds(wid * EPW + ci * CHUNK, CHUNK)],
                         sdidxA, semIA)
        compute(srowsA, edrowsA, orowsA)
        pltpu.make_async_copy(dsth.at[pl.ds(0, CHUNK)], sdidxA, semIA).wait()
        scatter(orowsA, sdidxA, semSA)

        @pl.when(p < NCHUNK // 2 - 1)
        def _():
            copy_idx(ci + 2, sdA)
            gather(sdA, srowsA, edrowsA, semGA)

        wait_gather(srowsB, edrowsB, semGB)

        @pl.when(p > 0)
        def _():
            wait_scatter(orowsB, semSB)

        pltpu.async_copy(dsth.at[pl.ds(wid * EPW + (ci + 1) * CHUNK, CHUNK)],
                         sdidxB, semIB)
        compute(srowsB, edrowsB, orowsB)
        pltpu.make_async_copy(dsth.at[pl.ds(0, CHUNK)], sdidxB, semIB).wait()
        scatter(orowsB, sdidxB, semSB)
        return c2

    lax.fori_loop(0, NCHUNK // 2, pair_body, 0, unroll=False)
    wait_scatter(orowsA, semSA)
    wait_scatter(orowsB, semSB)
    plsc.subcore_barrier()
    pltpu.sync_copy(acc.at[pl.ds(s * ROWS_PER_SUB, ROWS_PER_SUB)],
                    out_hbm.at[c, pl.ds(s * ROWS_PER_SUB, ROWS_PER_SUB)])


def _make_sc_edge(H):
    mesh = plsc.VectorSubcoreMesh(core_axis_name="c", subcore_axis_name="s")
    return functools.partial(
        pl.kernel,
        functools.partial(_sc_edge_body, H=H),
        mesh=mesh,
        compiler_params=pltpu.CompilerParams(use_tc_tiling_on_sc=False),
        out_type=jax.ShapeDtypeStruct((NCORES, NPAD, WIDTH), jnp.float32),
        scratch_types=[
            pltpu.VMEM((2 * CHUNK,), jnp.int32),
            pltpu.VMEM((2 * CHUNK,), jnp.int32),
            pltpu.VMEM((CHUNK,), jnp.int32),
            pltpu.VMEM((CHUNK,), jnp.int32),
            pltpu.VMEM((CHUNK, WIDTH), jnp.float32),
            pltpu.VMEM((CHUNK, WIDTH), jnp.float32),
            pltpu.VMEM((CHUNK, 16), jnp.float32),
            pltpu.VMEM((CHUNK, 16), jnp.float32),
            pltpu.VMEM((CHUNK, WIDTH), jnp.float32),
            pltpu.VMEM((CHUNK, WIDTH), jnp.float32),
            pltpu.VMEM((16,), jnp.float32),
            pltpu.VMEM_SHARED((NPAD, WIDTH), jnp.float32),
            pltpu.SemaphoreType.DMA,
            pltpu.SemaphoreType.DMA,
            pltpu.SemaphoreType.DMA,
            pltpu.SemaphoreType.DMA,
            pltpu.SemaphoreType.DMA,
            pltpu.SemaphoreType.DMA,
        ],
    )()


def _block_diag(a):
    H = a.shape[0]
    eye = jnp.eye(H, dtype=a.dtype)
    return (a[:, :, None] * eye[:, None, :]).reshape(H * a.shape[1], H)


def kernel(x, edge_index, indicator, n_norm, graph_index, W_emb, b_emb,
           W0, asrc0, adst0, W1, asrc1, adst1, W2, asrc2, adst2,
           W3, asrc3, adst3, Wm1, bm1, Wm2, bm2, Wm3, bm3):
    f32 = jnp.float32
    idxcat = edge_index.reshape(2, NW * NCHUNK, CHUNK).transpose(
        1, 0, 2).reshape(-1)
    dst = edge_index[1]
    zeros_pad = jnp.zeros((NPAD, WIDTH), f32)

    layer_W = [W0.reshape(D, D), W1.reshape(D, D), W2.reshape(D, D),
               W3.reshape(D, D)]
    layer_As = [_block_diag(asrc0), _block_diag(asrc1), _block_diag(asrc2),
                _block_diag(asrc3)]
    layer_Ad = [_block_diag(adst0), _block_diag(adst1), _block_diag(adst2),
                _block_diag(adst3)]
    R8 = jnp.kron(jnp.eye(8, dtype=f32), jnp.ones((1, 16), f32))
    R1 = jnp.ones((1, D), f32)

    table_shapes = (
        jax.ShapeDtypeStruct((N, D), f32),
        jax.ShapeDtypeStruct((NPAD, WIDTH), f32),
        jax.ShapeDtypeStruct((NPAD, 16), f32),
        jax.ShapeDtypeStruct((1, 16), f32),
    )

    h, srcT, edT, cvec = pl.pallas_call(
        _emb_prep_body, out_shape=table_shapes,
    )(x, W_emb, b_emb.reshape(1, D), layer_W[0], layer_As[0], layer_Ad[0])

    sc_edge8 = _make_sc_edge(8)
    sc_edge1 = _make_sc_edge(1)

    for i in range(3):
        part = (sc_edge8 if i < 3 else sc_edge1)(
            srcT, edT, cvec.reshape(16), idxcat, dst, zeros_pad)
        h, srcT, edT, cvec = pl.pallas_call(
            functools.partial(_finish_prep_body, H=8),
            out_shape=table_shapes,
        )(part, h, n_norm, R8, layer_W[i + 1], layer_As[i + 1],
          layer_Ad[i + 1])

    part = sc_edge1(srcT, edT, cvec.reshape(16), idxcat, dst, zeros_pad)
    y = pl.pallas_call(
        _finish_pool_body,
        out_shape=jax.ShapeDtypeStruct((N_GRAPHS, 10), f32),
    )(part, h, n_norm, R1, indicator.reshape(N, 1),
      Wm1, bm1.reshape(1, -1), Wm2, bm2.reshape(1, -1),
      Wm3, bm3.reshape(1, -1))
    return y
